# baseline (speedup 1.0000x reference)
import functools

import jax
import jax.numpy as jnp
from jax import lax
from jax.experimental import pallas as pl
from jax.experimental.pallas import tpu as pltpu
from jax.experimental.pallas import tpu_sc as plsc

NUM_TOKENS = 32768
MODEL_DIM = 1024
NUM_EXPERTS = 64
BLOCK_T = 1024
NUM_BLOCKS = NUM_TOKENS // BLOCK_T
NUM_STEPS = NUM_BLOCKS // 2

SC_TILES = 16
SC_CHUNK = NUM_TOKENS // SC_TILES
SC_GROUPS = SC_CHUNK // 16


def _half(x, w, eidx_f, gate_sel):
    E, B = NUM_EXPERTS, BLOCK_T
    lg = jax.lax.dot_general(
        w, x, (((1,), (1,)), ((), ())),
        preferred_element_type=jnp.float32)

    rowmax = jnp.max(lg, axis=0, keepdims=True)
    is_max = lg == rowmax
    idx_f = jnp.min(jnp.where(is_max, eidx_f, float(E)),
                    axis=0, keepdims=True)

    exps = jnp.exp(lg - rowmax)
    denom = jnp.sum(exps, axis=0, keepdims=True)
    gate = 1.0 / denom
    mask = (eidx_f == idx_f).astype(jnp.float32)

    ones_row = jnp.ones((1, B), jnp.float32)
    me_part = jax.lax.dot_general(
        gate, exps, (((1,), (1,)), ((), ())),
        preferred_element_type=jnp.float32)
    ce_part = jax.lax.dot_general(
        ones_row, mask, (((1,), (1,)), ((), ())),
        preferred_element_type=jnp.float32)
    del gate_sel
    return idx_f, gate, me_part, ce_part


def _gate_body(x0_ref, x1_ref, w_ref, eidx_ref,
               idx_ref, gate_ref, laux_ref, me_acc, cnt_acc):
    i = pl.program_id(0)

    @pl.when(i == 0)
    def _init():
        me_acc[...] = jnp.zeros_like(me_acc)
        cnt_acc[...] = jnp.zeros_like(cnt_acc)

    B = BLOCK_T
    w = w_ref[...]
    eidx_f = eidx_ref[...]

    idx0, gate0, me0, ce0 = _half(x0_ref[...], w, eidx_f, None)
    idx1, gate1, me1, ce1 = _half(x1_ref[...], w, eidx_f, None)

    idx_ref[0, 0, :] = idx0.astype(jnp.int32).reshape(B)
    idx_ref[0, 1, :] = idx1.astype(jnp.int32).reshape(B)
    gate_ref[0, 0, :] = gate0.reshape(B)
    gate_ref[0, 1, :] = gate1.reshape(B)
    me_acc[...] += me0 + me1
    cnt_acc[...] += ce0 + ce1

    @pl.when(i == NUM_STEPS - 1)
    def _fin():
        laux_ref[0, 0] = (jnp.sum(me_acc[...] * cnt_acc[...])
                          * (NUM_EXPERTS / (NUM_TOKENS * NUM_TOKENS)))


_SC_MESH = plsc.VectorSubcoreMesh(core_axis_name="c", subcore_axis_name="s")


@functools.partial(
    pl.kernel, mesh=_SC_MESH,
    out_type=jax.ShapeDtypeStruct((NUM_TOKENS,), jnp.int32),
    compiler_params=pltpu.CompilerParams(needs_layout_passes=False),
    scratch_types=[
        pltpu.VMEM((SC_CHUNK,), jnp.int32),
        pltpu.VMEM((SC_CHUNK,), jnp.int32),
        pltpu.VMEM((NUM_EXPERTS,), jnp.int32),
        pltpu.VMEM((NUM_EXPERTS,), jnp.int32),
        pltpu.VMEM((SC_TILES * NUM_EXPERTS,), jnp.int32),
        pltpu.VMEM_SHARED((SC_TILES * NUM_EXPERTS,), jnp.int32),
    ],
)
def _sc_locations(idx_hbm, out_hbm, idx_v, rank_v, cnt_v, off_v, hist_v,
                  shared):
    c = lax.axis_index("c")
    s = lax.axis_index("s")
    E = NUM_EXPERTS

    @pl.when(c == 0)
    def _count():
        wid = s
        base = wid * SC_CHUNK
        pltpu.sync_copy(idx_hbm.at[pl.ds(base, SC_CHUNK)], idx_v)

        for j in range(E // 16):
            cnt_v[pl.ds(j * 16, 16)] = jnp.zeros(16, jnp.int32)

        ones16 = jnp.ones(16, jnp.int32)
        for g in range(SC_GROUPS):
            v = idx_v[pl.ds(g * 16, 16)]
            cnt1, _ = plsc.scan_count(v.astype(jnp.float32))
            prior = plsc.load_gather(cnt_v, [v])
            rank_v[pl.ds(g * 16, 16)] = prior + cnt1.astype(jnp.int32) - 1
            plsc.addupdate_scatter(cnt_v, [v], ones16)

        pltpu.sync_copy(cnt_v, shared.at[pl.ds(wid * E, E)])

    plsc.subcore_barrier()
    plsc.subcore_barrier()

    @pl.when(c == 0)
    def _offset():
        wid = s
        base = wid * SC_CHUNK
        pltpu.sync_copy(shared, hist_v)

        for j in range(E // 16):
            off_v[pl.ds(j * 16, 16)] = jnp.zeros(16, jnp.int32)
        for w in range(SC_TILES):
            sel = (jnp.full((16,), w, jnp.int32)
                   < jnp.full((16,), 1, jnp.int32) * wid)
            for j in range(E // 16):
                off_v[pl.ds(j * 16, 16)] += jnp.where(
                    sel, hist_v[pl.ds(w * E + j * 16, 16)], 0)

        for g in range(SC_GROUPS):
            v = idx_v[pl.ds(g * 16, 16)]
            o = plsc.load_gather(off_v, [v])
            rank_v[pl.ds(g * 16, 16)] = rank_v[pl.ds(g * 16, 16)] + o

        pltpu.sync_copy(rank_v, out_hbm.at[pl.ds(base, SC_CHUNK)])


def kernel(input, W):
    num_tokens, num_experts = NUM_TOKENS, NUM_EXPERTS
    capacity = int((num_tokens + num_experts - 1) // num_experts)
    B = BLOCK_T

    row_i = jax.ShapeDtypeStruct((NUM_STEPS, 2, B), jnp.int32)
    row_f = jax.ShapeDtypeStruct((NUM_STEPS, 2, B), jnp.float32)
    pallas_fn = pl.pallas_call(
        _gate_body,
        grid=(NUM_STEPS,),
        in_specs=[
            pl.BlockSpec((B, MODEL_DIM), lambda i: (2 * i, 0)),
            pl.BlockSpec((B, MODEL_DIM), lambda i: (2 * i + 1, 0)),
            pl.BlockSpec((NUM_EXPERTS, MODEL_DIM), lambda i: (0, 0)),
            pl.BlockSpec((NUM_EXPERTS, B), lambda i: (0, 0)),
        ],
        out_specs=[
            pl.BlockSpec((1, 2, B), lambda i: (i, 0, 0)),
            pl.BlockSpec((1, 2, B), lambda i: (i, 0, 0)),
            pl.BlockSpec(memory_space=pltpu.SMEM),
        ],
        out_shape=[
            row_i, row_f,
            jax.ShapeDtypeStruct((1, 1), jnp.float32),
        ],
        scratch_shapes=[
            pltpu.VMEM((1, NUM_EXPERTS), jnp.float32),
            pltpu.VMEM((1, NUM_EXPERTS), jnp.float32),
        ],
    )

    eidx = jax.lax.broadcasted_iota(
        jnp.int32, (num_experts, B), 0).astype(jnp.float32)

    idx3, gate3, laux = pallas_fn(input, input, W, eidx)
    idx_flat = idx3.reshape(num_tokens)
    loc_flat = _sc_locations(idx_flat)
    return (laux[0, 0], idx_flat, capacity, loc_flat,
            gate3.reshape(num_tokens), num_experts)
